# baseline (speedup 1.0000x reference)
import functools

import jax
import jax.numpy as jnp
from jax.experimental import pallas as pl
from jax.experimental.pallas import tpu as pltpu
from jax.experimental.pallas import tpu_sc as plsc

N = 10000
D = 128
C16 = 16
NPAD = 10240
E = 320000
CHUNK = 128
NC, NS = 2, 16
NTILES = NC * NS
CPT = 80
NCHUNKS = NTILES * CPT
EPAD = NCHUNKS * CHUNK
CPT_F, CPT_S = 120, 40
PAIR = CPT_F + CPT_S
NCHUNKS_ALLOC = NCHUNKS + (CPT_F - CPT_S)
ROWS_PER_TILE = NPAD // NS
DUMMY_SRC = N
DUMMY_DST = N

_MESH = plsc.VectorSubcoreMesh(
    core_axis_name="c", subcore_axis_name="s", num_cores=NC, num_subcores=NS
)
_SC_PARAMS = pltpu.CompilerParams(use_tc_tiling_on_sc=False)


def _deg_kernel(dstp):
    @functools.partial(
        pl.kernel,
        out_type=jax.ShapeDtypeStruct((NC, NPAD, C16), jnp.float32),
        mesh=_MESH,
        scratch_types=[
            pltpu.VMEM((CPT, CHUNK), jnp.int32),
            pltpu.VMEM((2, CHUNK, C16), jnp.float32),
            pltpu.VMEM_SHARED((NPAD, C16), jnp.float32),
        ],
        compiler_params=_SC_PARAMS,
    )
    def deg(dst_hbm, out_hbm, didx, buf, acc):
        c = jax.lax.axis_index("c")
        s = jax.lax.axis_index("s")
        wid = s * NC + c

        @pl.loop(0, CHUNK)
        def _(r):
            buf[0, r, pl.ds(0, 16)] = jnp.zeros((16,), jnp.float32)
            buf[1, r, pl.ds(0, 16)] = jnp.ones((16,), jnp.float32)

        @pl.loop(0, ROWS_PER_TILE // CHUNK)
        def _(k):
            pltpu.sync_copy(
                buf.at[0], acc.at[pl.ds(s * ROWS_PER_TILE + k * CHUNK, CHUNK)]
            )

        plsc.subcore_barrier()
        pltpu.sync_copy(dst_hbm.at[pl.ds(wid * CPT, CPT)], didx)

        @pl.loop(0, CPT)
        def _(i):
            pltpu.sync_copy(buf.at[1], acc.at[didx.at[i]], add=True)

        plsc.subcore_barrier()
        pltpu.sync_copy(
            acc.at[pl.ds(s * ROWS_PER_TILE, ROWS_PER_TILE)],
            out_hbm.at[c].at[pl.ds(s * ROWS_PER_TILE, ROWS_PER_TILE)],
        )

    return deg(dstp)


def _make_agg(d):
    @functools.partial(
        pl.kernel,
        out_type=jax.ShapeDtypeStruct((NC, NPAD, d), jnp.float32),
        mesh=_MESH,
        scratch_types=[
            pltpu.VMEM((CPT_F, CHUNK), jnp.int32),
            pltpu.VMEM((CPT_F, CHUNK), jnp.int32),
            pltpu.VMEM((6, CHUNK, d), jnp.float32),
            pltpu.VMEM_SHARED((NPAD, d), jnp.float32),
            pltpu.SemaphoreType.DMA((6,)),
            pltpu.SemaphoreType.DMA((6,)),
        ],
        compiler_params=_SC_PARAMS,
    )
    def agg(hs_hbm, src_hbm, dst_hbm, out_hbm, sidx, didx, gbuf, acc, gsem, ssem):
        c = jax.lax.axis_index("c")
        s = jax.lax.axis_index("s")

        @pl.loop(0, CHUNK)
        def _(r):
            @pl.loop(0, d, step=16)
            def _(j):
                gbuf[0, r, pl.ds(j, 16)] = jnp.zeros((16,), jnp.float32)

        @pl.loop(0, ROWS_PER_TILE // CHUNK)
        def _(k):
            pltpu.sync_copy(
                gbuf.at[0], acc.at[pl.ds(s * ROWS_PER_TILE + k * CHUNK, CHUNK)]
            )

        plsc.subcore_barrier()
        base = s * PAIR + c * CPT_F
        nch = CPT_F - (CPT_F - CPT_S) * c
        pltpu.sync_copy(src_hbm.at[pl.ds(base, CPT_F)], sidx)
        pltpu.sync_copy(dst_hbm.at[pl.ds(base, CPT_F)], didx)

        G = 20
        NB = 6

        @pl.loop(0, nch, step=G)
        def _(base):
            gd = {}
            sd = {}
            for j in range(NB - 1):
                gd[j] = pltpu.async_copy(
                    hs_hbm.at[sidx.at[base + j]], gbuf.at[j], gsem.at[j])
            for j in range(G):
                s = j % NB
                gd.pop(s).wait()
                sd[j] = pltpu.async_copy(
                    gbuf.at[s], acc.at[didx.at[base + j]], ssem.at[s], add=True)
                nj = j + NB - 1
                if nj < G:
                    t = nj % NB
                    if (nj - NB) in sd:
                        sd.pop(nj - NB).wait()
                    gd[t] = pltpu.async_copy(
                        hs_hbm.at[sidx.at[base + nj]], gbuf.at[t], gsem.at[t])
            for j in sorted(sd):
                sd[j].wait()

        plsc.subcore_barrier()
        pltpu.sync_copy(
            acc.at[pl.ds(s * ROWS_PER_TILE, ROWS_PER_TILE)],
            out_hbm.at[c].at[pl.ds(s * ROWS_PER_TILE, ROWS_PER_TILE)],
        )

    return agg


DH = 64
_agg64 = _make_agg(DH)
_agg16 = _make_agg(C16)

_R = 1280
_PREC = jax.lax.Precision.HIGHEST


def _dinv_block(dp_ref):
    deg = 1.0 + dp_ref[0, :, 0:1] + dp_ref[1, :, 0:1]
    return jax.lax.rsqrt(deg)


def _row_mask(i):
    rows = jax.lax.broadcasted_iota(jnp.int32, (_R, 1), 0) + i * _R
    return rows < N


def _hs1_body(x_ref, w_ref, dp_ref, oa_ref, ob_ref):
    dinv = _dinv_block(dp_ref)
    h = jnp.dot(x_ref[...], w_ref[...], preferred_element_type=jnp.float32,
                precision=_PREC)
    hs = jnp.where(_row_mask(pl.program_id(0)), h * dinv, 0.0)
    oa_ref[...] = hs[:, :DH]
    ob_ref[...] = hs[:, DH:]


def _tc_hs1(x_pad, W1, degp):
    return pl.pallas_call(
        _hs1_body,
        grid=(NPAD // _R,),
        in_specs=[
            pl.BlockSpec((_R, D), lambda i: (i, 0)),
            pl.BlockSpec((D, D), lambda i: (0, 0)),
            pl.BlockSpec((NC, _R, C16), lambda i: (0, i, 0)),
        ],
        out_specs=[
            pl.BlockSpec((_R, DH), lambda i: (i, 0)),
            pl.BlockSpec((_R, DH), lambda i: (i, 0)),
        ],
        out_shape=[
            jax.ShapeDtypeStruct((NPAD, DH), jnp.float32),
            jax.ShapeDtypeStruct((NPAD, DH), jnp.float32),
        ],
    )(x_pad, W1, degp)


def _mid_body(pa_ref, pb_ref, ha_ref, hb_ref, dp_ref, w2_ref, b1_ref, o_ref):
    dinv = _dinv_block(dp_ref)
    agg = jnp.concatenate(
        [pa_ref[0] + pa_ref[1] + ha_ref[...],
         pb_ref[0] + pb_ref[1] + hb_ref[...]], axis=1)
    z = dinv * agg + b1_ref[...]
    r = jnp.maximum(z, 0.0)
    h2 = jnp.dot(r, w2_ref[...], preferred_element_type=jnp.float32,
                 precision=_PREC)
    o_ref[...] = jnp.where(_row_mask(pl.program_id(0)), h2 * dinv, 0.0)


def _tc_mid(pa, pb, hs1a, hs1b, degp, W2p, b1r):
    return pl.pallas_call(
        _mid_body,
        grid=(NPAD // _R,),
        in_specs=[
            pl.BlockSpec((NC, _R, DH), lambda i: (0, i, 0)),
            pl.BlockSpec((NC, _R, DH), lambda i: (0, i, 0)),
            pl.BlockSpec((_R, DH), lambda i: (i, 0)),
            pl.BlockSpec((_R, DH), lambda i: (i, 0)),
            pl.BlockSpec((NC, _R, C16), lambda i: (0, i, 0)),
            pl.BlockSpec((D, C16), lambda i: (0, 0)),
            pl.BlockSpec((1, D), lambda i: (0, 0)),
        ],
        out_specs=pl.BlockSpec((_R, C16), lambda i: (i, 0)),
        out_shape=jax.ShapeDtypeStruct((NPAD, C16), jnp.float32),
    )(pa, pb, hs1a, hs1b, degp, W2p, b1r)


def _out_body(q_ref, hs2_ref, dp_ref, b2_ref, o_ref):
    dinv = _dinv_block(dp_ref)
    o_ref[...] = dinv * (q_ref[0] + q_ref[1] + hs2_ref[...]) + b2_ref[...]


def _tc_out(q, hs2, degp, b2r):
    return pl.pallas_call(
        _out_body,
        grid=(NPAD // _R,),
        in_specs=[
            pl.BlockSpec((NC, _R, C16), lambda i: (0, i, 0)),
            pl.BlockSpec((_R, C16), lambda i: (i, 0)),
            pl.BlockSpec((NC, _R, C16), lambda i: (0, i, 0)),
            pl.BlockSpec((1, C16), lambda i: (0, 0)),
        ],
        out_specs=pl.BlockSpec((_R, C16), lambda i: (i, 0)),
        out_shape=jax.ShapeDtypeStruct((NPAD, C16), jnp.float32),
    )(q, hs2, degp, b2r)


@jax.jit
def kernel(x, edge_index, W1, b1, W2, b2):
    nclass = W2.shape[1]
    src = edge_index[0].astype(jnp.int32)
    dst = edge_index[1].astype(jnp.int32)
    pad = NCHUNKS_ALLOC * CHUNK - E
    srcp = jnp.concatenate(
        [src, jnp.full((pad,), DUMMY_SRC, jnp.int32)]).reshape(NCHUNKS_ALLOC, CHUNK)
    dstp = jnp.concatenate(
        [dst, jnp.full((pad,), DUMMY_DST, jnp.int32)]).reshape(NCHUNKS_ALLOC, CHUNK)
    x_pad = jnp.pad(x, ((0, NPAD - N), (0, 0)))
    b1r = b1.reshape(1, D)
    W2p = jnp.pad(W2, ((0, 0), (0, C16 - nclass)))
    b2r = jnp.pad(b2, (0, C16 - nclass)).reshape(1, C16)

    degp = _deg_kernel(dstp)
    hs1a, hs1b = _tc_hs1(x_pad, W1, degp)

    pa = _agg64(hs1a, srcp, dstp)
    pb = _agg64(hs1b, srcp, dstp)
    hs2 = _tc_mid(pa, pb, hs1a, hs1b, degp, W2p, b1r)
    q = _agg16(hs2, srcp, dstp)
    out = _tc_out(q, hs2, degp, b2r)
    return out[:N, :nclass]


# speedup vs baseline: 1.7857x; 1.7857x over previous
import functools

import jax
import jax.numpy as jnp
from jax.experimental import pallas as pl
from jax.experimental.pallas import tpu as pltpu
from jax.experimental.pallas import tpu_sc as plsc

N = 10000
D = 128
C16 = 16
NPAD = 10240
E = 320000
CHUNK = 128
NC, NS = 2, 16
NTILES = NC * NS
CPT = 80
NCHUNKS = NTILES * CPT
EPAD = NCHUNKS * CHUNK
CPT_F, CPT_S = 80, 80
PAIR = CPT_F + CPT_S
NCHUNKS_ALLOC = NCHUNKS + (CPT_F - CPT_S)
ROWS_PER_TILE = NPAD // NS
DUMMY_SRC = N
DUMMY_DST = N

_MESH = plsc.VectorSubcoreMesh(
    core_axis_name="c", subcore_axis_name="s", num_cores=NC, num_subcores=NS
)
_SC_PARAMS = pltpu.CompilerParams(use_tc_tiling_on_sc=False)


def _deg_kernel(dstp):
    @functools.partial(
        pl.kernel,
        out_type=jax.ShapeDtypeStruct((NC, NPAD, C16), jnp.float32),
        mesh=_MESH,
        scratch_types=[
            pltpu.VMEM((CPT, CHUNK), jnp.int32),
            pltpu.VMEM((2, CHUNK, C16), jnp.float32),
            pltpu.VMEM_SHARED((NPAD, C16), jnp.float32),
        ],
        compiler_params=_SC_PARAMS,
    )
    def deg(dst_hbm, out_hbm, didx, buf, acc):
        c = jax.lax.axis_index("c")
        s = jax.lax.axis_index("s")
        wid = s * NC + c

        @pl.loop(0, CHUNK)
        def _(r):
            buf[0, r, pl.ds(0, 16)] = jnp.zeros((16,), jnp.float32)
            buf[1, r, pl.ds(0, 16)] = jnp.ones((16,), jnp.float32)

        @pl.loop(0, ROWS_PER_TILE // CHUNK)
        def _(k):
            pltpu.sync_copy(
                buf.at[0], acc.at[pl.ds(s * ROWS_PER_TILE + k * CHUNK, CHUNK)]
            )

        plsc.subcore_barrier()
        pltpu.sync_copy(dst_hbm.at[pl.ds(wid * CPT, CPT)], didx)

        @pl.loop(0, CPT)
        def _(i):
            pltpu.sync_copy(buf.at[1], acc.at[didx.at[i]], add=True)

        plsc.subcore_barrier()
        pltpu.sync_copy(
            acc.at[pl.ds(s * ROWS_PER_TILE, ROWS_PER_TILE)],
            out_hbm.at[c].at[pl.ds(s * ROWS_PER_TILE, ROWS_PER_TILE)],
        )

    return deg(dstp)


def _make_agg(d):
    @functools.partial(
        pl.kernel,
        out_type=jax.ShapeDtypeStruct((NC, NPAD, d), jnp.float32),
        mesh=_MESH,
        scratch_types=[
            pltpu.VMEM((CPT_F, CHUNK), jnp.int32),
            pltpu.VMEM((CPT_F, CHUNK), jnp.int32),
            pltpu.VMEM((2, CHUNK, d), jnp.float32),
            pltpu.VMEM_SHARED((NPAD, d), jnp.float32),
            pltpu.VMEM_SHARED((NPAD, d), jnp.float32),
            pltpu.SemaphoreType.DMA((2,)),
            pltpu.SemaphoreType.DMA((2,)),
        ],
        compiler_params=_SC_PARAMS,
    )
    def agg(hs_hbm, src_hbm, dst_hbm, out_hbm, sidx, didx, gbuf, acc, hs_spm, gsem, ssem):
        c = jax.lax.axis_index("c")
        s = jax.lax.axis_index("s")

        @pl.loop(0, CHUNK)
        def _(r):
            @pl.loop(0, d, step=16)
            def _(j):
                gbuf[0, r, pl.ds(j, 16)] = jnp.zeros((16,), jnp.float32)

        @pl.loop(0, ROWS_PER_TILE // CHUNK)
        def _(k):
            pltpu.sync_copy(
                gbuf.at[0], acc.at[pl.ds(s * ROWS_PER_TILE + k * CHUNK, CHUNK)]
            )

        pltpu.sync_copy(
            hs_hbm.at[pl.ds(s * ROWS_PER_TILE, ROWS_PER_TILE)],
            hs_spm.at[pl.ds(s * ROWS_PER_TILE, ROWS_PER_TILE)],
        )
        plsc.subcore_barrier()
        base = s * PAIR + c * CPT_F
        nch = CPT_F
        pltpu.sync_copy(src_hbm.at[pl.ds(base, CPT_F)], sidx)
        pltpu.sync_copy(dst_hbm.at[pl.ds(base, CPT_F)], didx)

        G = 20
        NB = 2

        @pl.loop(0, nch, step=G)
        def _(base):
            gd = {}
            sd = {}
            for j in range(NB - 1):
                gd[j] = pltpu.async_copy(
                    hs_spm.at[sidx.at[base + j]], gbuf.at[j], gsem.at[j])
            for j in range(G):
                s = j % NB
                gd.pop(s).wait()
                sd[j] = pltpu.async_copy(
                    gbuf.at[s], acc.at[didx.at[base + j]], ssem.at[s], add=True)
                nj = j + NB - 1
                if nj < G:
                    t = nj % NB
                    if (nj - NB) in sd:
                        sd.pop(nj - NB).wait()
                    gd[t] = pltpu.async_copy(
                        hs_spm.at[sidx.at[base + nj]], gbuf.at[t], gsem.at[t])
            for j in sorted(sd):
                sd[j].wait()

        plsc.subcore_barrier()
        pltpu.sync_copy(
            acc.at[pl.ds(s * ROWS_PER_TILE, ROWS_PER_TILE)],
            out_hbm.at[c].at[pl.ds(s * ROWS_PER_TILE, ROWS_PER_TILE)],
        )

    return agg


DH = 64
_agg64 = _make_agg(DH)
_agg16 = _make_agg(C16)

_R = 1280
_PREC = jax.lax.Precision.HIGHEST


def _dinv_block(dp_ref):
    deg = 1.0 + dp_ref[0, :, 0:1] + dp_ref[1, :, 0:1]
    return jax.lax.rsqrt(deg)


def _row_mask(i):
    rows = jax.lax.broadcasted_iota(jnp.int32, (_R, 1), 0) + i * _R
    return rows < N


def _hs1_body(x_ref, w_ref, dp_ref, oa_ref, ob_ref):
    dinv = _dinv_block(dp_ref)
    h = jnp.dot(x_ref[...], w_ref[...], preferred_element_type=jnp.float32,
                precision=_PREC)
    hs = jnp.where(_row_mask(pl.program_id(0)), h * dinv, 0.0)
    oa_ref[...] = hs[:, :DH]
    ob_ref[...] = hs[:, DH:]


def _tc_hs1(x_pad, W1, degp):
    return pl.pallas_call(
        _hs1_body,
        grid=(NPAD // _R,),
        in_specs=[
            pl.BlockSpec((_R, D), lambda i: (i, 0)),
            pl.BlockSpec((D, D), lambda i: (0, 0)),
            pl.BlockSpec((NC, _R, C16), lambda i: (0, i, 0)),
        ],
        out_specs=[
            pl.BlockSpec((_R, DH), lambda i: (i, 0)),
            pl.BlockSpec((_R, DH), lambda i: (i, 0)),
        ],
        out_shape=[
            jax.ShapeDtypeStruct((NPAD, DH), jnp.float32),
            jax.ShapeDtypeStruct((NPAD, DH), jnp.float32),
        ],
    )(x_pad, W1, degp)


def _mid_body(pa_ref, pb_ref, ha_ref, hb_ref, dp_ref, w2_ref, b1_ref, o_ref):
    dinv = _dinv_block(dp_ref)
    agg = jnp.concatenate(
        [pa_ref[0] + pa_ref[1] + ha_ref[...],
         pb_ref[0] + pb_ref[1] + hb_ref[...]], axis=1)
    z = dinv * agg + b1_ref[...]
    r = jnp.maximum(z, 0.0)
    h2 = jnp.dot(r, w2_ref[...], preferred_element_type=jnp.float32,
                 precision=_PREC)
    o_ref[...] = jnp.where(_row_mask(pl.program_id(0)), h2 * dinv, 0.0)


def _tc_mid(pa, pb, hs1a, hs1b, degp, W2p, b1r):
    return pl.pallas_call(
        _mid_body,
        grid=(NPAD // _R,),
        in_specs=[
            pl.BlockSpec((NC, _R, DH), lambda i: (0, i, 0)),
            pl.BlockSpec((NC, _R, DH), lambda i: (0, i, 0)),
            pl.BlockSpec((_R, DH), lambda i: (i, 0)),
            pl.BlockSpec((_R, DH), lambda i: (i, 0)),
            pl.BlockSpec((NC, _R, C16), lambda i: (0, i, 0)),
            pl.BlockSpec((D, C16), lambda i: (0, 0)),
            pl.BlockSpec((1, D), lambda i: (0, 0)),
        ],
        out_specs=pl.BlockSpec((_R, C16), lambda i: (i, 0)),
        out_shape=jax.ShapeDtypeStruct((NPAD, C16), jnp.float32),
    )(pa, pb, hs1a, hs1b, degp, W2p, b1r)


def _out_body(q_ref, hs2_ref, dp_ref, b2_ref, o_ref):
    dinv = _dinv_block(dp_ref)
    o_ref[...] = dinv * (q_ref[0] + q_ref[1] + hs2_ref[...]) + b2_ref[...]


def _tc_out(q, hs2, degp, b2r):
    return pl.pallas_call(
        _out_body,
        grid=(NPAD // _R,),
        in_specs=[
            pl.BlockSpec((NC, _R, C16), lambda i: (0, i, 0)),
            pl.BlockSpec((_R, C16), lambda i: (i, 0)),
            pl.BlockSpec((NC, _R, C16), lambda i: (0, i, 0)),
            pl.BlockSpec((1, C16), lambda i: (0, 0)),
        ],
        out_specs=pl.BlockSpec((_R, C16), lambda i: (i, 0)),
        out_shape=jax.ShapeDtypeStruct((NPAD, C16), jnp.float32),
    )(q, hs2, degp, b2r)


@jax.jit
def kernel(x, edge_index, W1, b1, W2, b2):
    nclass = W2.shape[1]
    src = edge_index[0].astype(jnp.int32)
    dst = edge_index[1].astype(jnp.int32)
    pad = NCHUNKS_ALLOC * CHUNK - E
    srcp = jnp.concatenate(
        [src, jnp.full((pad,), DUMMY_SRC, jnp.int32)]).reshape(NCHUNKS_ALLOC, CHUNK)
    dstp = jnp.concatenate(
        [dst, jnp.full((pad,), DUMMY_DST, jnp.int32)]).reshape(NCHUNKS_ALLOC, CHUNK)
    x_pad = jnp.pad(x, ((0, NPAD - N), (0, 0)))
    b1r = b1.reshape(1, D)
    W2p = jnp.pad(W2, ((0, 0), (0, C16 - nclass)))
    b2r = jnp.pad(b2, (0, C16 - nclass)).reshape(1, C16)

    degp = _deg_kernel(dstp)
    hs1a, hs1b = _tc_hs1(x_pad, W1, degp)

    pa = _agg64(hs1a, srcp, dstp)
    pb = _agg64(hs1b, srcp, dstp)
    hs2 = _tc_mid(pa, pb, hs1a, hs1b, degp, W2p, b1r)
    q = _agg16(hs2, srcp, dstp)
    out = _tc_out(q, hs2, degp, b2r)
    return out[:N, :nclass]


# speedup vs baseline: 1.9014x; 1.0648x over previous
import functools

import jax
import jax.numpy as jnp
from jax.experimental import pallas as pl
from jax.experimental.pallas import tpu as pltpu
from jax.experimental.pallas import tpu_sc as plsc

N = 10000
D = 128
C16 = 16
NPAD = 10240
E = 320000
CHUNK = 128
NC, NS = 2, 16
NTILES = NC * NS
CPT = 80
NCHUNKS = NTILES * CPT
EPAD = NCHUNKS * CHUNK
CPT_F, CPT_S = 80, 80
PAIR = CPT_F + CPT_S
NCHUNKS_ALLOC = NCHUNKS + (CPT_F - CPT_S)
ROWS_PER_TILE = NPAD // NS
DUMMY_SRC = N
DUMMY_DST = N

_MESH = plsc.VectorSubcoreMesh(
    core_axis_name="c", subcore_axis_name="s", num_cores=NC, num_subcores=NS
)
_SC_PARAMS = pltpu.CompilerParams(use_tc_tiling_on_sc=False)


def _deg_kernel(dstp):
    @functools.partial(
        pl.kernel,
        out_type=jax.ShapeDtypeStruct((NC, NPAD, C16), jnp.float32),
        mesh=_MESH,
        scratch_types=[
            pltpu.VMEM((CPT, CHUNK), jnp.int32),
            pltpu.VMEM((2, CHUNK, C16), jnp.float32),
            pltpu.VMEM_SHARED((NPAD, C16), jnp.float32),
        ],
        compiler_params=_SC_PARAMS,
    )
    def deg(dst_hbm, out_hbm, didx, buf, acc):
        c = jax.lax.axis_index("c")
        s = jax.lax.axis_index("s")
        wid = s * NC + c

        @pl.loop(0, CHUNK)
        def _(r):
            buf[0, r, pl.ds(0, 16)] = jnp.zeros((16,), jnp.float32)
            buf[1, r, pl.ds(0, 16)] = jnp.ones((16,), jnp.float32)

        @pl.loop(0, ROWS_PER_TILE // CHUNK)
        def _(k):
            pltpu.sync_copy(
                buf.at[0], acc.at[pl.ds(s * ROWS_PER_TILE + k * CHUNK, CHUNK)]
            )

        plsc.subcore_barrier()
        pltpu.sync_copy(dst_hbm.at[pl.ds(wid * CPT, CPT)], didx)

        @pl.loop(0, CPT)
        def _(i):
            pltpu.sync_copy(buf.at[1], acc.at[didx.at[i]], add=True)

        plsc.subcore_barrier()
        pltpu.sync_copy(
            acc.at[pl.ds(s * ROWS_PER_TILE, ROWS_PER_TILE)],
            out_hbm.at[c].at[pl.ds(s * ROWS_PER_TILE, ROWS_PER_TILE)],
        )

    return deg(dstp)


def _zero_init(gbuf, acc, s, d):
    @pl.loop(0, CHUNK)
    def _(r):
        @pl.loop(0, d, step=16)
        def _(j):
            gbuf[0, r, pl.ds(j, 16)] = jnp.zeros((16,), jnp.float32)

    @pl.loop(0, ROWS_PER_TILE // CHUNK)
    def _(k):
        pltpu.sync_copy(
            gbuf.at[0], acc.at[pl.ds(s * ROWS_PER_TILE + k * CHUNK, CHUNK)]
        )


def _pipe_group(hs_spm, acc, sidx, didx, gbuf, gsem, ssem, g, nb):
    gd = {}
    sd = {}
    for j in range(nb - 1):
        gd[j] = pltpu.async_copy(hs_spm.at[sidx.at[j]], gbuf.at[j], gsem.at[j])
    for j in range(g):
        b = j % nb
        gd.pop(b).wait()
        sd[j] = pltpu.async_copy(
            gbuf.at[b], acc.at[didx.at[j]], ssem.at[b], add=True)
        nj = j + nb - 1
        if nj < g:
            t = nj % nb
            if (nj - nb) in sd:
                sd.pop(nj - nb).wait()
            gd[t] = pltpu.async_copy(
                hs_spm.at[sidx.at[nj]], gbuf.at[t], gsem.at[t])
    for j in sorted(sd):
        sd[j].wait()


DH = 64
GF = 40


def _agg_feat(hsab, srcp, dstp):
    @functools.partial(
        pl.kernel,
        out_type=jax.ShapeDtypeStruct((NC, NPAD, DH), jnp.float32),
        mesh=_MESH,
        scratch_types=[
            pltpu.VMEM((GF, CHUNK), jnp.int32),
            pltpu.VMEM((GF, CHUNK), jnp.int32),
            pltpu.VMEM((2, CHUNK, DH), jnp.float32),
            pltpu.VMEM_SHARED((NPAD, DH), jnp.float32),
            pltpu.VMEM_SHARED((NPAD, DH), jnp.float32),
            pltpu.SemaphoreType.DMA((2,)),
            pltpu.SemaphoreType.DMA((2,)),
        ],
        compiler_params=_SC_PARAMS,
    )
    def agg(hs_hbm, src_hbm, dst_hbm, out_hbm, sidx, didx, gbuf, acc, hs_spm,
            gsem, ssem):
        c = jax.lax.axis_index("c")
        s = jax.lax.axis_index("s")
        _zero_init(gbuf, acc, s, DH)
        pltpu.sync_copy(
            hs_hbm.at[c].at[pl.ds(s * ROWS_PER_TILE, ROWS_PER_TILE)],
            hs_spm.at[pl.ds(s * ROWS_PER_TILE, ROWS_PER_TILE)],
        )
        plsc.subcore_barrier()

        @pl.loop(0, PAIR, step=GF)
        def _(g):
            base = s * PAIR + g
            pltpu.sync_copy(src_hbm.at[pl.ds(base, GF)], sidx)
            pltpu.sync_copy(dst_hbm.at[pl.ds(base, GF)], didx)
            _pipe_group(hs_spm, acc, sidx, didx, gbuf, gsem, ssem, GF, 2)

        plsc.subcore_barrier()
        pltpu.sync_copy(
            acc.at[pl.ds(s * ROWS_PER_TILE, ROWS_PER_TILE)],
            out_hbm.at[c].at[pl.ds(s * ROWS_PER_TILE, ROWS_PER_TILE)],
        )

    return agg(hsab, srcp, dstp)


def _make_agg(d, nb):
    @functools.partial(
        pl.kernel,
        out_type=jax.ShapeDtypeStruct((NC, NPAD, d), jnp.float32),
        mesh=_MESH,
        scratch_types=[
            pltpu.VMEM((CPT_F, CHUNK), jnp.int32),
            pltpu.VMEM((CPT_F, CHUNK), jnp.int32),
            pltpu.VMEM((nb, CHUNK, d), jnp.float32),
            pltpu.VMEM_SHARED((NPAD, d), jnp.float32),
            pltpu.VMEM_SHARED((NPAD, d), jnp.float32),
            pltpu.SemaphoreType.DMA((nb,)),
            pltpu.SemaphoreType.DMA((nb,)),
        ],
        compiler_params=_SC_PARAMS,
    )
    def agg(hs_hbm, src_hbm, dst_hbm, out_hbm, sidx, didx, gbuf, acc, hs_spm,
            gsem, ssem):
        c = jax.lax.axis_index("c")
        s = jax.lax.axis_index("s")
        _zero_init(gbuf, acc, s, d)
        pltpu.sync_copy(
            hs_hbm.at[pl.ds(s * ROWS_PER_TILE, ROWS_PER_TILE)],
            hs_spm.at[pl.ds(s * ROWS_PER_TILE, ROWS_PER_TILE)],
        )
        plsc.subcore_barrier()
        base = s * PAIR + c * CPT_F
        pltpu.sync_copy(src_hbm.at[pl.ds(base, CPT_F)], sidx)
        pltpu.sync_copy(dst_hbm.at[pl.ds(base, CPT_F)], didx)

        G = 20

        @pl.loop(0, CPT_F, step=G)
        def _(gbase):
            _pipe_group(hs_spm, acc,
                        sidx.at[pl.ds(gbase, G)], didx.at[pl.ds(gbase, G)],
                        gbuf, gsem, ssem, G, nb)

        plsc.subcore_barrier()
        pltpu.sync_copy(
            acc.at[pl.ds(s * ROWS_PER_TILE, ROWS_PER_TILE)],
            out_hbm.at[c].at[pl.ds(s * ROWS_PER_TILE, ROWS_PER_TILE)],
        )

    return agg


_agg16 = _make_agg(C16, 4)

_R = 1280
_PREC = jax.lax.Precision.HIGHEST


def _dinv_block(dp_ref):
    deg = 1.0 + dp_ref[0, :, 0:1] + dp_ref[1, :, 0:1]
    return jax.lax.rsqrt(deg)


def _row_mask(i):
    rows = jax.lax.broadcasted_iota(jnp.int32, (_R, 1), 0) + i * _R
    return rows < N


def _hs1_body(x_ref, w_ref, dp_ref, o_ref):
    dinv = _dinv_block(dp_ref)
    h = jnp.dot(x_ref[...], w_ref[...], preferred_element_type=jnp.float32,
                precision=_PREC)
    hs = jnp.where(_row_mask(pl.program_id(0)), h * dinv, 0.0)
    o_ref[0] = hs[:, :DH]
    o_ref[1] = hs[:, DH:]


def _tc_hs1(x_pad, W1, degp):
    return pl.pallas_call(
        _hs1_body,
        grid=(NPAD // _R,),
        in_specs=[
            pl.BlockSpec((_R, D), lambda i: (i, 0)),
            pl.BlockSpec((D, D), lambda i: (0, 0)),
            pl.BlockSpec((NC, _R, C16), lambda i: (0, i, 0)),
        ],
        out_specs=pl.BlockSpec((NC, _R, DH), lambda i: (0, i, 0)),
        out_shape=jax.ShapeDtypeStruct((NC, NPAD, DH), jnp.float32),
    )(x_pad, W1, degp)


def _mid_body(p_ref, h_ref, dp_ref, w2_ref, b1_ref, o_ref):
    dinv = _dinv_block(dp_ref)
    agg = jnp.concatenate(
        [p_ref[0] + h_ref[0], p_ref[1] + h_ref[1]], axis=1)
    z = dinv * agg + b1_ref[...]
    r = jnp.maximum(z, 0.0)
    h2 = jnp.dot(r, w2_ref[...], preferred_element_type=jnp.float32,
                 precision=_PREC)
    o_ref[...] = jnp.where(_row_mask(pl.program_id(0)), h2 * dinv, 0.0)


def _tc_mid(p, hsab, degp, W2p, b1r):
    return pl.pallas_call(
        _mid_body,
        grid=(NPAD // _R,),
        in_specs=[
            pl.BlockSpec((NC, _R, DH), lambda i: (0, i, 0)),
            pl.BlockSpec((NC, _R, DH), lambda i: (0, i, 0)),
            pl.BlockSpec((NC, _R, C16), lambda i: (0, i, 0)),
            pl.BlockSpec((D, C16), lambda i: (0, 0)),
            pl.BlockSpec((1, D), lambda i: (0, 0)),
        ],
        out_specs=pl.BlockSpec((_R, C16), lambda i: (i, 0)),
        out_shape=jax.ShapeDtypeStruct((NPAD, C16), jnp.float32),
    )(p, hsab, degp, W2p, b1r)


def _out_body(q_ref, hs2_ref, dp_ref, b2_ref, o_ref):
    dinv = _dinv_block(dp_ref)
    o_ref[...] = dinv * (q_ref[0] + q_ref[1] + hs2_ref[...]) + b2_ref[...]


def _tc_out(q, hs2, degp, b2r):
    return pl.pallas_call(
        _out_body,
        grid=(NPAD // _R,),
        in_specs=[
            pl.BlockSpec((NC, _R, C16), lambda i: (0, i, 0)),
            pl.BlockSpec((_R, C16), lambda i: (i, 0)),
            pl.BlockSpec((NC, _R, C16), lambda i: (0, i, 0)),
            pl.BlockSpec((1, C16), lambda i: (0, 0)),
        ],
        out_specs=pl.BlockSpec((_R, C16), lambda i: (i, 0)),
        out_shape=jax.ShapeDtypeStruct((NPAD, C16), jnp.float32),
    )(q, hs2, degp, b2r)


@jax.jit
def kernel(x, edge_index, W1, b1, W2, b2):
    nclass = W2.shape[1]
    src = edge_index[0].astype(jnp.int32)
    dst = edge_index[1].astype(jnp.int32)
    pad = NCHUNKS_ALLOC * CHUNK - E
    srcp = jnp.concatenate(
        [src, jnp.full((pad,), DUMMY_SRC, jnp.int32)]).reshape(NCHUNKS_ALLOC, CHUNK)
    dstp = jnp.concatenate(
        [dst, jnp.full((pad,), DUMMY_DST, jnp.int32)]).reshape(NCHUNKS_ALLOC, CHUNK)
    x_pad = jnp.pad(x, ((0, NPAD - N), (0, 0)))
    b1r = b1.reshape(1, D)
    W2p = jnp.pad(W2, ((0, 0), (0, C16 - nclass)))
    b2r = jnp.pad(b2, (0, C16 - nclass)).reshape(1, C16)

    degp = _deg_kernel(dstp)
    hsab = _tc_hs1(x_pad, W1, degp)
    p = _agg_feat(hsab, srcp, dstp)
    hs2 = _tc_mid(p, hsab, degp, W2p, b1r)
    q = _agg16(hs2, srcp, dstp)
    out = _tc_out(q, hs2, degp, b2r)
    return out[:N, :nclass]


# speedup vs baseline: 1.9775x; 1.0400x over previous
import functools

import jax
import jax.numpy as jnp
from jax.experimental import pallas as pl
from jax.experimental.pallas import tpu as pltpu
from jax.experimental.pallas import tpu_sc as plsc

N = 10000
D = 128
C16 = 16
NPAD = 10240
E = 320000
CHUNK = 128
NC, NS = 2, 16
NTILES = NC * NS
NCHUNKS = E // CHUNK
GSZ = 20
NGROUPS = NCHUNKS // GSZ
ROWS_PER_TILE = NPAD // NS

_MESH = plsc.VectorSubcoreMesh(
    core_axis_name="c", subcore_axis_name="s", num_cores=NC, num_subcores=NS
)
_SC_PARAMS = pltpu.CompilerParams(use_tc_tiling_on_sc=False)


def _deg_kernel(eidx):
    @functools.partial(
        pl.kernel,
        out_type=jax.ShapeDtypeStruct((NC, NPAD, C16), jnp.float32),
        mesh=_MESH,
        scratch_types=[
            pltpu.VMEM((GSZ, CHUNK), jnp.int32),
            pltpu.VMEM((2, CHUNK, C16), jnp.float32),
            pltpu.VMEM_SHARED((NPAD, C16), jnp.float32),
        ],
        compiler_params=_SC_PARAMS,
    )
    def deg(eidx_hbm, out_hbm, didx, buf, acc):
        c = jax.lax.axis_index("c")
        s = jax.lax.axis_index("s")
        wid = s * NC + c

        @pl.loop(0, CHUNK)
        def _(r):
            buf[0, r, pl.ds(0, 16)] = jnp.zeros((16,), jnp.float32)
            buf[1, r, pl.ds(0, 16)] = jnp.ones((16,), jnp.float32)

        @pl.loop(0, ROWS_PER_TILE // CHUNK)
        def _(k):
            pltpu.sync_copy(
                buf.at[0], acc.at[pl.ds(s * ROWS_PER_TILE + k * CHUNK, CHUNK)]
            )

        plsc.subcore_barrier()

        @pl.loop(wid, NGROUPS, step=NTILES)
        def _(g):
            pltpu.sync_copy(eidx_hbm.at[1].at[pl.ds(g * GSZ, GSZ)], didx)

            @pl.loop(0, GSZ)
            def _(i):
                pltpu.sync_copy(buf.at[1], acc.at[didx.at[i]], add=True)

        plsc.subcore_barrier()
        pltpu.sync_copy(
            acc.at[pl.ds(s * ROWS_PER_TILE, ROWS_PER_TILE)],
            out_hbm.at[c].at[pl.ds(s * ROWS_PER_TILE, ROWS_PER_TILE)],
        )

    return deg(eidx)


def _zero_init(gbuf, acc, s, d):
    @pl.loop(0, CHUNK)
    def _(r):
        @pl.loop(0, d, step=16)
        def _(j):
            gbuf[0, r, pl.ds(j, 16)] = jnp.zeros((16,), jnp.float32)

    @pl.loop(0, ROWS_PER_TILE // CHUNK)
    def _(k):
        pltpu.sync_copy(
            gbuf.at[0], acc.at[pl.ds(s * ROWS_PER_TILE + k * CHUNK, CHUNK)]
        )


def _pipe_group(hs_spm, acc, sidx, didx, gbuf, gsem, ssem, g, nb):
    gd = {}
    sd = {}
    for j in range(nb - 1):
        gd[j] = pltpu.async_copy(hs_spm.at[sidx.at[j]], gbuf.at[j], gsem.at[j])
    for j in range(g):
        b = j % nb
        gd.pop(b).wait()
        sd[j] = pltpu.async_copy(
            gbuf.at[b], acc.at[didx.at[j]], ssem.at[b], add=True)
        nj = j + nb - 1
        if nj < g:
            t = nj % nb
            if (nj - nb) in sd:
                sd.pop(nj - nb).wait()
            gd[t] = pltpu.async_copy(
                hs_spm.at[sidx.at[nj]], gbuf.at[t], gsem.at[t])
    for j in sorted(sd):
        sd[j].wait()


DH = 64


def _agg_feat(hsab, eidx):
    @functools.partial(
        pl.kernel,
        out_type=jax.ShapeDtypeStruct((NC, NPAD, DH), jnp.float32),
        mesh=_MESH,
        scratch_types=[
            pltpu.VMEM((GSZ, CHUNK), jnp.int32),
            pltpu.VMEM((GSZ, CHUNK), jnp.int32),
            pltpu.VMEM((2, CHUNK, DH), jnp.float32),
            pltpu.VMEM_SHARED((NPAD, DH), jnp.float32),
            pltpu.VMEM_SHARED((NPAD, DH), jnp.float32),
            pltpu.SemaphoreType.DMA((2,)),
            pltpu.SemaphoreType.DMA((2,)),
        ],
        compiler_params=_SC_PARAMS,
    )
    def agg(hs_hbm, eidx_hbm, out_hbm, sidx, didx, gbuf, acc, hs_spm,
            gsem, ssem):
        c = jax.lax.axis_index("c")
        s = jax.lax.axis_index("s")
        _zero_init(gbuf, acc, s, DH)
        pltpu.sync_copy(
            hs_hbm.at[c].at[pl.ds(s * ROWS_PER_TILE, ROWS_PER_TILE)],
            hs_spm.at[pl.ds(s * ROWS_PER_TILE, ROWS_PER_TILE)],
        )
        plsc.subcore_barrier()

        @pl.loop(s, NGROUPS, step=NS)
        def _(g):
            pltpu.sync_copy(eidx_hbm.at[0].at[pl.ds(g * GSZ, GSZ)], sidx)
            pltpu.sync_copy(eidx_hbm.at[1].at[pl.ds(g * GSZ, GSZ)], didx)
            _pipe_group(hs_spm, acc, sidx, didx, gbuf, gsem, ssem, GSZ, 2)

        plsc.subcore_barrier()
        pltpu.sync_copy(
            acc.at[pl.ds(s * ROWS_PER_TILE, ROWS_PER_TILE)],
            out_hbm.at[c].at[pl.ds(s * ROWS_PER_TILE, ROWS_PER_TILE)],
        )

    return agg(hsab, eidx)


def _make_agg(d, nb):
    @functools.partial(
        pl.kernel,
        out_type=jax.ShapeDtypeStruct((NC, NPAD, d), jnp.float32),
        mesh=_MESH,
        scratch_types=[
            pltpu.VMEM((GSZ, CHUNK), jnp.int32),
            pltpu.VMEM((GSZ, CHUNK), jnp.int32),
            pltpu.VMEM((nb, CHUNK, d), jnp.float32),
            pltpu.VMEM_SHARED((NPAD, d), jnp.float32),
            pltpu.VMEM_SHARED((NPAD, d), jnp.float32),
            pltpu.SemaphoreType.DMA((nb,)),
            pltpu.SemaphoreType.DMA((nb,)),
        ],
        compiler_params=_SC_PARAMS,
    )
    def agg(hs_hbm, eidx_hbm, out_hbm, sidx, didx, gbuf, acc, hs_spm,
            gsem, ssem):
        c = jax.lax.axis_index("c")
        s = jax.lax.axis_index("s")
        wid = s * NC + c
        _zero_init(gbuf, acc, s, d)
        pltpu.sync_copy(
            hs_hbm.at[pl.ds(s * ROWS_PER_TILE, ROWS_PER_TILE)],
            hs_spm.at[pl.ds(s * ROWS_PER_TILE, ROWS_PER_TILE)],
        )
        plsc.subcore_barrier()

        @pl.loop(wid, NGROUPS, step=NTILES)
        def _(g):
            pltpu.sync_copy(eidx_hbm.at[0].at[pl.ds(g * GSZ, GSZ)], sidx)
            pltpu.sync_copy(eidx_hbm.at[1].at[pl.ds(g * GSZ, GSZ)], didx)
            _pipe_group(hs_spm, acc, sidx, didx, gbuf, gsem, ssem, GSZ, nb)

        plsc.subcore_barrier()
        pltpu.sync_copy(
            acc.at[pl.ds(s * ROWS_PER_TILE, ROWS_PER_TILE)],
            out_hbm.at[c].at[pl.ds(s * ROWS_PER_TILE, ROWS_PER_TILE)],
        )

    return agg


_agg16 = _make_agg(C16, 4)

_R = 1280
_PREC = jax.lax.Precision.HIGHEST


def _dinv_block(dp_ref):
    deg = 1.0 + dp_ref[0, :, 0:1] + dp_ref[1, :, 0:1]
    return jax.lax.rsqrt(deg)


def _row_mask(i):
    rows = jax.lax.broadcasted_iota(jnp.int32, (_R, 1), 0) + i * _R
    return rows < N


def _mm_body(x_ref, w_ref, o_ref):
    o_ref[...] = jnp.dot(x_ref[...], w_ref[...],
                         preferred_element_type=jnp.float32, precision=_PREC)


def _tc_mm(x_pad, W1):
    return pl.pallas_call(
        _mm_body,
        grid=(NPAD // _R,),
        in_specs=[
            pl.BlockSpec((_R, D), lambda i: (i, 0)),
            pl.BlockSpec((D, D), lambda i: (0, 0)),
        ],
        out_specs=pl.BlockSpec((_R, D), lambda i: (i, 0)),
        out_shape=jax.ShapeDtypeStruct((NPAD, D), jnp.float32),
    )(x_pad, W1)


def _hs1_body(h_ref, dp_ref, o_ref):
    dinv = _dinv_block(dp_ref)
    hs = jnp.where(_row_mask(pl.program_id(0)), h_ref[...] * dinv, 0.0)
    o_ref[0] = hs[:, :DH]
    o_ref[1] = hs[:, DH:]


def _tc_hs1(h, degp):
    return pl.pallas_call(
        _hs1_body,
        grid=(NPAD // _R,),
        in_specs=[
            pl.BlockSpec((_R, D), lambda i: (i, 0)),
            pl.BlockSpec((NC, _R, C16), lambda i: (0, i, 0)),
        ],
        out_specs=pl.BlockSpec((NC, _R, DH), lambda i: (0, i, 0)),
        out_shape=jax.ShapeDtypeStruct((NC, NPAD, DH), jnp.float32),
    )(h, degp)


def _mid_body(p_ref, h_ref, dp_ref, w2_ref, b1_ref, o_ref):
    dinv = _dinv_block(dp_ref)
    agg = jnp.concatenate(
        [p_ref[0] + h_ref[0], p_ref[1] + h_ref[1]], axis=1)
    z = dinv * agg + b1_ref[...]
    r = jnp.maximum(z, 0.0)
    h2 = jnp.dot(r, w2_ref[...], preferred_element_type=jnp.float32,
                 precision=_PREC)
    o_ref[...] = jnp.where(_row_mask(pl.program_id(0)), h2 * dinv, 0.0)


def _tc_mid(p, hsab, degp, W2p, b1r):
    return pl.pallas_call(
        _mid_body,
        grid=(NPAD // _R,),
        in_specs=[
            pl.BlockSpec((NC, _R, DH), lambda i: (0, i, 0)),
            pl.BlockSpec((NC, _R, DH), lambda i: (0, i, 0)),
            pl.BlockSpec((NC, _R, C16), lambda i: (0, i, 0)),
            pl.BlockSpec((D, C16), lambda i: (0, 0)),
            pl.BlockSpec((1, D), lambda i: (0, 0)),
        ],
        out_specs=pl.BlockSpec((_R, C16), lambda i: (i, 0)),
        out_shape=jax.ShapeDtypeStruct((NPAD, C16), jnp.float32),
    )(p, hsab, degp, W2p, b1r)


def _out_body(q_ref, hs2_ref, dp_ref, b2_ref, o_ref):
    dinv = _dinv_block(dp_ref)
    o_ref[...] = dinv * (q_ref[0] + q_ref[1] + hs2_ref[...]) + b2_ref[...]


def _tc_out(q, hs2, degp, b2r):
    return pl.pallas_call(
        _out_body,
        grid=(NPAD // _R,),
        in_specs=[
            pl.BlockSpec((NC, _R, C16), lambda i: (0, i, 0)),
            pl.BlockSpec((_R, C16), lambda i: (i, 0)),
            pl.BlockSpec((NC, _R, C16), lambda i: (0, i, 0)),
            pl.BlockSpec((1, C16), lambda i: (0, 0)),
        ],
        out_specs=pl.BlockSpec((_R, C16), lambda i: (i, 0)),
        out_shape=jax.ShapeDtypeStruct((NPAD, C16), jnp.float32),
    )(q, hs2, degp, b2r)


@jax.jit
def kernel(x, edge_index, W1, b1, W2, b2):
    nclass = W2.shape[1]
    eidx = edge_index.astype(jnp.int32).reshape(2, NCHUNKS, CHUNK)
    x_pad = jnp.pad(x, ((0, NPAD - N), (0, 0)))
    b1r = b1.reshape(1, D)
    W2p = jnp.pad(W2, ((0, 0), (0, C16 - nclass)))
    b2r = jnp.pad(b2, (0, C16 - nclass)).reshape(1, C16)

    degp = _deg_kernel(eidx)
    h = _tc_mm(x_pad, W1)
    hsab = _tc_hs1(h, degp)
    p = _agg_feat(hsab, eidx)
    hs2 = _tc_mid(p, hsab, degp, W2p, b1r)
    q = _agg16(hs2, eidx)
    out = _tc_out(q, hs2, degp, b2r)
    return out[:N, :nclass]
